# baseline (speedup 1.0000x reference)
import functools

import jax
import jax.numpy as jnp
from jax import lax
from jax.experimental import pallas as pl
from jax.experimental.pallas import tpu as pltpu
from jax.experimental.pallas import tpu_sc as plsc

NC = 2
NS = 16
L = 16
NT = NC * NS
CH = 128


def _mesh():
    return plsc.VectorSubcoreMesh(
        core_axis_name="c", subcore_axis_name="s", num_cores=NC, num_subcores=NS
    )


def _wid():
    return lax.axis_index("s") * NC + lax.axis_index("c")


def _k1_body(ept, ncht, n, rows_ref, cols_ref, tab_ref, row_v, col_v, key2,
             ids2):
    tbase = _wid() * ept
    pltpu.sync_copy(rows_ref.at[pl.ds(tbase, ept)], row_v)
    pltpu.sync_copy(cols_ref.at[pl.ds(tbase, ept)], col_v)
    lane = lax.iota(jnp.int32, L)

    def compute(i, c):
        for k in range(CH // L):
            off = i * CH + k * L
            r = row_v[pl.ds(off, L)]
            cc = col_v[pl.ds(off, L)]
            key2[i, pl.ds(k * L, L)] = r * n + cc
            ids2[i, pl.ds(k * L, L)] = (tbase + off) + lane
        return c

    lax.fori_loop(0, ncht, compute, 0)

    def scat(i, c):
        pltpu.sync_copy(ids2.at[i], tab_ref.at[key2.at[i]])
        return c

    lax.fori_loop(0, ncht, scat, 0)


def _k2_body(ept, ncht, n, npad, rows_ref, cols_ref, tab_ref, hist_ref,
             row2_ref, col2_ref, row_v, col_v, key2, ids2, w2, keep2, ro2,
             rr2, co2, zb, hist_s):
    cid = lax.axis_index("c")
    sid = lax.axis_index("s")
    wid = sid * NC + cid
    tbase = wid * ept
    slc = npad // NS

    def z(i, c):
        zb[pl.ds(i * L, L)] = jnp.zeros((L,), jnp.float32)
        return c

    lax.fori_loop(0, slc // L, z, 0)
    pltpu.sync_copy(zb, hist_s.at[pl.ds(sid * slc, slc)])
    plsc.subcore_barrier()

    pltpu.sync_copy(rows_ref.at[pl.ds(tbase, ept)], row_v)
    pltpu.sync_copy(cols_ref.at[pl.ds(tbase, ept)], col_v)
    lane = lax.iota(jnp.int32, L)

    def compute(i, c):
        for k in range(CH // L):
            sl = pl.ds(k * L, L)
            off = i * CH + k * L
            r = row_v[pl.ds(off, L)]
            cc = col_v[pl.ds(off, L)]
            key2[i, sl] = r * n + cc
            ids2[i, sl] = (tbase + off) + lane
            ro2[i, sl] = r
            co2[i, sl] = cc
        return c

    lax.fori_loop(0, ncht, compute, 0)

    def chunk(i, c):
        pltpu.sync_copy(tab_ref.at[key2.at[i]], w2.at[i])
        for k in range(CH // L):
            sl = pl.ds(k * L, L)
            keep = w2[i, sl] == ids2[i, sl]
            keep2[i, sl] = jnp.where(keep, 1.0, 0.0).astype(jnp.float32)
            rr2[i, sl] = jnp.where(keep, ro2[i, sl], n)
        pltpu.sync_copy(keep2.at[i], hist_s.at[ro2.at[i]], add=True)
        return c

    lax.fori_loop(0, ncht, chunk, 0)

    cbase = wid * ncht
    pltpu.sync_copy(rr2, row2_ref.at[pl.ds(cbase, ncht)])
    pltpu.sync_copy(co2, col2_ref.at[pl.ds(cbase, ncht)])
    plsc.subcore_barrier()
    pltpu.sync_copy(hist_s.at[pl.ds(sid * slc, slc)],
                    hist_ref.at[pl.ds(cid * npad + sid * slc, slc)])


def _k4_body(ncht, npad, dim, col2_ref, row2_ref, t_ref, acc_ref,
             co2, rr2, buf, zb2, acc_s):
    cid = lax.axis_index("c")
    sid = lax.axis_index("s")
    wid = sid * NC + cid
    slc = npad // NS

    def z(i, c):
        for k in range(dim // L):
            zb2[i, pl.ds(k * L, L)] = jnp.zeros((L,), jnp.float32)
        return c

    lax.fori_loop(0, L, z, 0)

    def zs(j, c):
        pltpu.sync_copy(zb2, acc_s.at[pl.ds(sid * slc + j * L, L)])
        return c

    lax.fori_loop(0, slc // L, zs, 0)
    plsc.subcore_barrier()

    cbase = wid * ncht
    pltpu.sync_copy(col2_ref.at[pl.ds(cbase, ncht)], co2)
    pltpu.sync_copy(row2_ref.at[pl.ds(cbase, ncht)], rr2)

    def chunk(i, c):
        pltpu.sync_copy(t_ref.at[co2.at[i]], buf)
        pltpu.sync_copy(buf, acc_s.at[rr2.at[i]], add=True)
        return c

    lax.fori_loop(0, ncht, chunk, 0)
    plsc.subcore_barrier()
    pltpu.sync_copy(acc_s.at[pl.ds(sid * slc, slc)],
                    acc_ref.at[cid, pl.ds(sid * slc, slc)])


def _k3_body(x_ref, w_ref, h_ref, f_ref, s_ref, t_ref):
    s = jnp.dot(x_ref[...], w_ref[...], preferred_element_type=jnp.float32)
    h = h_ref[...]
    deg = h[0] + h[1] + 1.0
    f = f_ref[...]
    d = f * (lax.rsqrt(deg) - 1.0) + 1.0
    s_ref[...] = s
    t_ref[...] = d * s


def _k5_body(acc_ref, h_ref, s_ref, b_ref, f_ref, o_ref):
    acc = acc_ref[...]
    h = h_ref[...]
    deg = h[0] + h[1] + 1.0
    f = f_ref[...]
    d = f * (lax.rsqrt(deg) - 1.0) + 1.0
    agg = acc[0] + acc[1]
    o_ref[...] = d * (agg + d * s_ref[...]) + b_ref[...]


def kernel(input, edge_index, need_norm, weight, bias):
    x = input.astype(jnp.float32)
    n, d_in = x.shape
    d_out = weight.shape[1]
    e = edge_index.shape[1]

    npad = ((n + 1 + 255) // 256) * 256
    blk = 128
    ngrid = npad // blk

    quantum = NT * CH * 8
    epad = ((e + quantum - 1) // quantum) * quantum
    ei = edge_index.astype(jnp.int32)
    if epad != e:
        pad = jnp.broadcast_to(ei[:, :1], (2, epad - e))
        ei = jnp.concatenate([ei, pad], axis=1)
    rows = ei[0]
    cols = ei[1]
    ept = epad // NT
    ncht = ept // CH

    f32 = jnp.float32
    i32 = jnp.int32

    k1 = pl.kernel(
        functools.partial(_k1_body, ept, ncht, n),
        out_type=jax.ShapeDtypeStruct((n * n,), i32),
        mesh=_mesh(),
        scratch_types=[
            pltpu.VMEM((ept,), i32),
            pltpu.VMEM((ept,), i32),
            pltpu.VMEM((ncht, CH), i32),
            pltpu.VMEM((ncht, CH), i32),
        ],
    )
    tab = k1(rows, cols)

    k2 = pl.kernel(
        functools.partial(_k2_body, ept, ncht, n, npad),
        out_type=(
            jax.ShapeDtypeStruct((2 * npad,), f32),
            jax.ShapeDtypeStruct((epad // CH, CH), i32),
            jax.ShapeDtypeStruct((epad // CH, CH), i32),
        ),
        mesh=_mesh(),
        scratch_types=[
            pltpu.VMEM((ept,), i32),
            pltpu.VMEM((ept,), i32),
            pltpu.VMEM((ncht, CH), i32),
            pltpu.VMEM((ncht, CH), i32),
            pltpu.VMEM((ncht, CH), i32),
            pltpu.VMEM((ncht, CH), f32),
            pltpu.VMEM((ncht, CH), i32),
            pltpu.VMEM((ncht, CH), i32),
            pltpu.VMEM((ncht, CH), i32),
            pltpu.VMEM((npad // NS,), f32),
            pltpu.VMEM_SHARED((npad,), f32),
        ],
    )
    hist, row2, col2 = k2(rows, cols, tab)

    xp = jnp.pad(x, ((0, npad - n), (0, 0)))
    hist3 = hist.reshape(2, npad, 1)
    flag = (need_norm != 0).astype(f32).reshape(1, 1)
    s, t = pl.pallas_call(
        _k3_body,
        grid=(ngrid,),
        in_specs=[
            pl.BlockSpec((blk, d_in), lambda i: (i, 0)),
            pl.BlockSpec((d_in, d_out), lambda i: (0, 0)),
            pl.BlockSpec((2, blk, 1), lambda i: (0, i, 0)),
            pl.BlockSpec((1, 1), lambda i: (0, 0)),
        ],
        out_specs=[
            pl.BlockSpec((blk, d_out), lambda i: (i, 0)),
            pl.BlockSpec((blk, d_out), lambda i: (i, 0)),
        ],
        out_shape=[
            jax.ShapeDtypeStruct((npad, d_out), f32),
            jax.ShapeDtypeStruct((npad, d_out), f32),
        ],
    )(xp, weight.astype(f32), hist3, flag)

    k4 = pl.kernel(
        functools.partial(_k4_body, ncht, npad, d_out),
        out_type=jax.ShapeDtypeStruct((2, npad, d_out), f32),
        mesh=_mesh(),
        scratch_types=[
            pltpu.VMEM((ncht, CH), i32),
            pltpu.VMEM((ncht, CH), i32),
            pltpu.VMEM((CH, d_out), f32),
            pltpu.VMEM((L, d_out), f32),
            pltpu.VMEM_SHARED((npad, d_out), f32),
        ],
    )
    acc = k4(col2, row2, t)

    out = pl.pallas_call(
        _k5_body,
        grid=(ngrid,),
        in_specs=[
            pl.BlockSpec((2, blk, d_out), lambda i: (0, i, 0)),
            pl.BlockSpec((2, blk, 1), lambda i: (0, i, 0)),
            pl.BlockSpec((blk, d_out), lambda i: (i, 0)),
            pl.BlockSpec((1, d_out), lambda i: (0, 0)),
            pl.BlockSpec((1, 1), lambda i: (0, 0)),
        ],
        out_specs=pl.BlockSpec((blk, d_out), lambda i: (i, 0)),
        out_shape=jax.ShapeDtypeStruct((npad, d_out), f32),
    )(acc, hist3, s, bias.astype(f32).reshape(1, d_out), flag)

    return out[:n]
